# baseline (speedup 1.0000x reference)
import functools
import math

import jax
import jax.numpy as jnp
from jax import lax
from jax.experimental import pallas as pl
from jax.experimental.pallas import tpu as pltpu
from jax.experimental.pallas import tpu_sc as plsc

NINP = 128
SEQ = 200
BATCH = 1024
ROWS = SEQ * BATCH
NWORKERS = 32
RPW = ROWS // NWORKERS
STAGE = 256
NSTAGES = RPW // STAGE
RING = 3
AHEAD = 2
SROWS = 16
LANES = 16
_SCALE = math.sqrt(NINP)


def _pe_table():
    position = jnp.arange(0, SEQ, dtype=jnp.float32)[:, None]
    div_term = jnp.exp(
        jnp.arange(0, NINP, 2, dtype=jnp.float32) * (-math.log(10000.0) / NINP)
    )
    pe = jnp.zeros((SEQ, NINP), dtype=jnp.float32)
    pe = pe.at[:, 0::2].set(jnp.sin(position * div_term))
    pe = pe.at[:, 1::2].set(jnp.cos(position * div_term))
    return pe


@functools.partial(
    pl.kernel,
    mesh=plsc.VectorSubcoreMesh(core_axis_name="c", subcore_axis_name="s"),
    out_type=jax.ShapeDtypeStruct((ROWS, NINP), jnp.float32),
    scratch_types=[
        pltpu.VMEM((RPW,), jnp.int32),
        pltpu.VMEM((SROWS, NINP), jnp.float32),
        pltpu.VMEM((STAGE, NINP), jnp.float32),
        pltpu.VMEM((STAGE, NINP), jnp.float32),
        pltpu.VMEM((STAGE, NINP), jnp.float32),
        pltpu.SemaphoreType.DMA,
        pltpu.SemaphoreType.DMA,
        pltpu.SemaphoreType.DMA,
        pltpu.SemaphoreType.DMA,
        pltpu.SemaphoreType.DMA,
        pltpu.SemaphoreType.DMA,
    ],
)
def _encode_sc(idx_hbm, table_hbm, pe_hbm, out_hbm,
               idx_v, pe_v, b0, b1, b2,
               gsem0, gsem1, gsem2,
               wsem0, wsem1, wsem2):
    wid = lax.axis_index("s") * 2 + lax.axis_index("c")
    base = wid * RPW
    s_lo = base // BATCH
    s_lo8 = pl.multiple_of(
        jnp.minimum(s_lo - lax.rem(s_lo, 8), SEQ - SROWS), 8)

    pltpu.sync_copy(idx_hbm.at[pl.ds(base, RPW)], idx_v)
    pltpu.sync_copy(pe_hbm.at[pl.ds(s_lo8, SROWS)], pe_v)

    bufs = (b0, b1, b2)
    gsems = (gsem0, gsem1, gsem2)
    wsems = (wsem0, wsem1, wsem2)

    def start_gather(t, b):
        pltpu.async_copy(
            table_hbm.at[idx_v.at[pl.ds(t * STAGE, STAGE)]], bufs[b],
            gsems[b])

    def wait_gather(b):
        pltpu.make_async_copy(
            table_hbm.at[pl.ds(0, STAGE)], bufs[b], gsems[b]).wait()

    def wait_write(b):
        pltpu.make_async_copy(
            bufs[b], out_hbm.at[pl.ds(0, STAGE)], wsems[b]).wait()

    def compute(t, b):
        buf = bufs[b]
        ds = (base + t * STAGE) // BATCH - s_lo8
        pe_rows = [pe_v[ds, pl.ds(LANES * j, LANES)]
                   for j in range(NINP // LANES)]

        def row(r, carry):
            for j in range(NINP // LANES):
                sl = pl.ds(LANES * j, LANES)
                buf[r, sl] = buf[r, sl] * _SCALE + pe_rows[j]
            return carry

        lax.fori_loop(0, STAGE, row, 0)

    for t in range(AHEAD):
        start_gather(t, t)

    def stage_body(i, p, t):
        b = p
        wait_gather(b)
        compute(t, b)
        pltpu.async_copy(bufs[b], out_hbm.at[pl.ds(base + t * STAGE, STAGE)],
                         wsems[b])
        bn = (p + AHEAD) % RING

        @pl.when(t + AHEAD < NSTAGES)
        def _():
            if p >= RING - AHEAD:
                wait_write(bn)
            else:
                @pl.when(i > 0)
                def _():
                    wait_write(bn)
            start_gather(t + AHEAD, bn)

    def outer(i, carry):
        for p in range(RING):
            stage_body(i, p, RING * i + p)
        return carry

    lax.fori_loop(0, NSTAGES // RING, outer, 0)
    for p in range(NSTAGES % RING):
        t = (NSTAGES // RING) * RING + p
        wait_gather(p)
        compute(t, p)
        pltpu.async_copy(bufs[p], out_hbm.at[pl.ds(base + t * STAGE, STAGE)],
                         wsems[p])
    for b in range(RING):
        wait_write(b)


def kernel(src, emb_weight):
    idx = src.T.reshape(-1).astype(jnp.int32)
    out = _encode_sc(idx, emb_weight, _pe_table())
    return out.reshape(SEQ, BATCH, NINP)


# speedup vs baseline: 1.0127x; 1.0127x over previous
import functools
import math

import jax
import jax.numpy as jnp
from jax import lax
from jax.experimental import pallas as pl
from jax.experimental.pallas import tpu as pltpu
from jax.experimental.pallas import tpu_sc as plsc

NINP = 128
SEQ = 200
BATCH = 1024
ROWS = SEQ * BATCH
NWORKERS = 32
RPW = ROWS // NWORKERS
STAGE = 128
NSTAGES = RPW // STAGE
NG = 5
NO = 2
NPH = 10
SROWS = 16
LANES = 16
_SCALE = math.sqrt(NINP)


def _pe_table():
    position = jnp.arange(0, SEQ, dtype=jnp.float32)[:, None]
    div_term = jnp.exp(
        jnp.arange(0, NINP, 2, dtype=jnp.float32) * (-math.log(10000.0) / NINP)
    )
    pe = jnp.zeros((SEQ, NINP), dtype=jnp.float32)
    pe = pe.at[:, 0::2].set(jnp.sin(position * div_term))
    pe = pe.at[:, 1::2].set(jnp.cos(position * div_term))
    return pe


@functools.partial(
    pl.kernel,
    mesh=plsc.VectorSubcoreMesh(core_axis_name="c", subcore_axis_name="s"),
    out_type=jax.ShapeDtypeStruct((ROWS, NINP), jnp.float32),
    scratch_types=[
        pltpu.VMEM((RPW,), jnp.int32),
        pltpu.VMEM((SROWS, NINP), jnp.float32),
        pltpu.VMEM((STAGE, NINP), jnp.float32),
        pltpu.VMEM((STAGE, NINP), jnp.float32),
        pltpu.VMEM((STAGE, NINP), jnp.float32),
        pltpu.VMEM((STAGE, NINP), jnp.float32),
        pltpu.VMEM((STAGE, NINP), jnp.float32),
        pltpu.VMEM((STAGE, NINP), jnp.float32),
        pltpu.VMEM((STAGE, NINP), jnp.float32),
        pltpu.SemaphoreType.DMA,
        pltpu.SemaphoreType.DMA,
        pltpu.SemaphoreType.DMA,
        pltpu.SemaphoreType.DMA,
        pltpu.SemaphoreType.DMA,
        pltpu.SemaphoreType.DMA,
        pltpu.SemaphoreType.DMA,
    ],
)
def _encode_sc(idx_hbm, table_hbm, pe_hbm, out_hbm,
               idx_v, pe_v, g0, g1, g2, g3, g4, o0, o1,
               gsem0, gsem1, gsem2, gsem3, gsem4, wsem0, wsem1):
    wid = lax.axis_index("s") * 2 + lax.axis_index("c")
    base = wid * RPW
    s_lo = base // BATCH
    s_lo8 = pl.multiple_of(
        jnp.minimum(s_lo - lax.rem(s_lo, 8), SEQ - SROWS), 8)

    pltpu.sync_copy(idx_hbm.at[pl.ds(base, RPW)], idx_v)
    pltpu.sync_copy(pe_hbm.at[pl.ds(s_lo8, SROWS)], pe_v)

    gbufs = (g0, g1, g2, g3, g4)
    gsems = (gsem0, gsem1, gsem2, gsem3, gsem4)
    obufs = (o0, o1)
    wsems = (wsem0, wsem1)

    def start_gather(t, b):
        pltpu.async_copy(
            table_hbm.at[idx_v.at[pl.ds(t * STAGE, STAGE)]], gbufs[b],
            gsems[b])

    def wait_gather(b):
        pltpu.make_async_copy(
            table_hbm.at[pl.ds(0, STAGE)], gbufs[b], gsems[b]).wait()

    def wait_write(b):
        pltpu.make_async_copy(
            obufs[b], out_hbm.at[pl.ds(0, STAGE)], wsems[b]).wait()

    def compute(t, gb, ob):
        gbuf, obuf = gbufs[gb], obufs[ob]
        ds = (base + t * STAGE) // BATCH - s_lo8
        pe_rows = [pe_v[ds, pl.ds(LANES * j, LANES)]
                   for j in range(NINP // LANES)]

        def row(r, carry):
            for j in range(NINP // LANES):
                sl = pl.ds(LANES * j, LANES)
                obuf[r, sl] = gbuf[r, sl] * _SCALE + pe_rows[j]
            return carry

        lax.fori_loop(0, STAGE, row, 0)

    for t in range(NG - 1):
        start_gather(t, t)

    def stage_body(i, p, t):
        gb, ob = p % NG, p % NO
        wait_gather(gb)
        if p >= NO:
            wait_write(ob)
        else:
            @pl.when(i > 0)
            def _():
                wait_write(ob)

        @pl.when(t + NG - 1 < NSTAGES)
        def _():
            start_gather(t + NG - 1, (p + NG - 1) % NG)

        compute(t, gb, ob)
        pltpu.async_copy(obufs[ob],
                         out_hbm.at[pl.ds(base + t * STAGE, STAGE)],
                         wsems[ob])

    def outer(i, carry):
        for p in range(NPH):
            stage_body(i, p, NPH * i + p)
        return carry

    lax.fori_loop(0, NSTAGES // NPH, outer, 0)
    for b in range(NO):
        wait_write(b)


def kernel(src, emb_weight):
    idx = src.T.reshape(-1).astype(jnp.int32)
    out = _encode_sc(idx, emb_weight, _pe_table())
    return out.reshape(SEQ, BATCH, NINP)


# speedup vs baseline: 1.0184x; 1.0056x over previous
import functools
import math

import jax
import jax.numpy as jnp
from jax import lax
from jax.experimental import pallas as pl
from jax.experimental.pallas import tpu as pltpu
from jax.experimental.pallas import tpu_sc as plsc

NINP = 128
SEQ = 200
BATCH = 1024
ROWS = SEQ * BATCH
NWORKERS = 32
RPW = ROWS // NWORKERS
STAGE = 128
NSTAGES = RPW // STAGE
NPHASE = 3
SROWS = 16
LANES = 16
_SCALE = math.sqrt(NINP)


def _pe_table():
    position = jnp.arange(0, SEQ, dtype=jnp.float32)[:, None]
    div_term = jnp.exp(
        jnp.arange(0, NINP, 2, dtype=jnp.float32) * (-math.log(10000.0) / NINP)
    )
    pe = jnp.zeros((SEQ, NINP), dtype=jnp.float32)
    pe = pe.at[:, 0::2].set(jnp.sin(position * div_term))
    pe = pe.at[:, 1::2].set(jnp.cos(position * div_term))
    return pe


@functools.partial(
    pl.kernel,
    mesh=plsc.VectorSubcoreMesh(core_axis_name="c", subcore_axis_name="s"),
    out_type=jax.ShapeDtypeStruct((ROWS, NINP), jnp.float32),
    scratch_types=[
        pltpu.VMEM((RPW,), jnp.int32),
        pltpu.VMEM((SROWS, NINP), jnp.float32),
        pltpu.VMEM((STAGE, NINP), jnp.float32),
        pltpu.VMEM((STAGE, NINP), jnp.float32),
        pltpu.VMEM((STAGE, NINP), jnp.float32),
        pltpu.VMEM((STAGE, NINP), jnp.float32),
        pltpu.VMEM((STAGE, NINP), jnp.float32),
        pltpu.VMEM((STAGE, NINP), jnp.float32),
        pltpu.SemaphoreType.DMA,
        pltpu.SemaphoreType.DMA,
        pltpu.SemaphoreType.DMA,
        pltpu.SemaphoreType.DMA,
        pltpu.SemaphoreType.DMA,
        pltpu.SemaphoreType.DMA,
    ],
)
def _encode_sc(idx_hbm, table_hbm, pe_hbm, out_hbm,
               idx_v, pe_v, g0, g1, g2, o0, o1, o2,
               gsem0, gsem1, gsem2, wsem0, wsem1, wsem2):
    wid = lax.axis_index("s") * 2 + lax.axis_index("c")
    base = wid * RPW
    s_lo = base // BATCH
    s_lo8 = pl.multiple_of(
        jnp.minimum(s_lo - lax.rem(s_lo, 8), SEQ - SROWS), 8)

    pltpu.sync_copy(idx_hbm.at[pl.ds(base, RPW)], idx_v)
    pltpu.sync_copy(pe_hbm.at[pl.ds(s_lo8, SROWS)], pe_v)

    def start_gather(t, gbuf, gsem):
        pltpu.async_copy(
            table_hbm.at[idx_v.at[pl.ds(t * STAGE, STAGE)]], gbuf, gsem)

    def wait_gather(gbuf, gsem):
        pltpu.make_async_copy(table_hbm.at[pl.ds(0, STAGE)], gbuf, gsem).wait()

    def wait_write(obuf, wsem):
        pltpu.make_async_copy(obuf, out_hbm.at[pl.ds(0, STAGE)], wsem).wait()

    def compute(t, gbuf, obuf):
        ds = (base + t * STAGE) // BATCH - s_lo8
        pe_rows = [pe_v[ds, pl.ds(LANES * j, LANES)]
                   for j in range(NINP // LANES)]

        def row(r, carry):
            for j in range(NINP // LANES):
                sl = pl.ds(LANES * j, LANES)
                obuf[r, sl] = gbuf[r, sl] * _SCALE + pe_rows[j]
            return carry

        lax.fori_loop(0, STAGE, row, 0)

    phases = ((g0, o0, gsem0, wsem0), (g1, o1, gsem1, wsem1),
              (g2, o2, gsem2, wsem2))

    for t in range(NPHASE):
        start_gather(t, phases[t][0], phases[t][2])

    def outer(i, carry):
        for p, (gbuf, obuf, gsem, wsem) in enumerate(phases):
            t = NPHASE * i + p
            wait_gather(gbuf, gsem)

            @pl.when(i > 0)
            def _():
                wait_write(obuf, wsem)

            compute(t, gbuf, obuf)

            @pl.when(t + NPHASE < NSTAGES)
            def _():
                start_gather(t + NPHASE, gbuf, gsem)

            pltpu.async_copy(obuf, out_hbm.at[pl.ds(base + t * STAGE, STAGE)],
                             wsem)
        return carry

    lax.fori_loop(0, NSTAGES // NPHASE, outer, 0)
    for p in range(NSTAGES % NPHASE):
        t = (NSTAGES // NPHASE) * NPHASE + p
        gbuf, obuf, gsem, wsem = phases[p]
        wait_gather(gbuf, gsem)
        wait_write(obuf, wsem)
        compute(t, gbuf, obuf)
        pltpu.async_copy(obuf, out_hbm.at[pl.ds(base + t * STAGE, STAGE)],
                         wsem)
    for p, (gbuf, obuf, gsem, wsem) in enumerate(phases):
        wait_write(obuf, wsem)


def kernel(src, emb_weight):
    idx = src.T.reshape(-1).astype(jnp.int32)
    out = _encode_sc(idx, emb_weight, _pe_table())
    return out.reshape(SEQ, BATCH, NINP)
